# baseline (speedup 1.0000x reference)
import jax
import jax.numpy as jnp
from jax import lax
from jax.experimental import pallas as pl
from jax.experimental.pallas import tpu as pltpu
from jax.experimental.pallas import tpu_sc as plsc

B = 1024
HIDDEN = 4096
DEPTH = 8
EPS = 1e-5

BM = 512
BK = 256
BM2 = 64
N_TILES = 32
RPT = B // N_TILES
CHUNK = 8
TRASH = B
B_PAD = B + 8


def _k1_body(h_ref, w_ref, bp_ref, g_ref, be_ref, wg_ref, bg_ref, ptr_ref,
             top_ref, nptr_ref, wslot_ref, gsafe_ref, tgt_ref,
             acc_ref, gacc_ref):
    k = pl.program_id(1)
    nk = pl.num_programs(1)

    h_bf = h_ref[...].astype(jnp.bfloat16)
    w_bf = w_ref[...].astype(jnp.bfloat16)
    part = jnp.dot(h_bf, w_bf, preferred_element_type=jnp.float32)
    wg_bf = wg_ref[...].astype(jnp.bfloat16)
    gpart = jnp.dot(h_bf, wg_bf, preferred_element_type=jnp.float32)

    @pl.when(k == 0)
    def _():
        acc_ref[...] = part
        gacc_ref[...] = gpart

    @pl.when(k != 0)
    def _():
        acc_ref[...] += part
        gacc_ref[...] += gpart

    @pl.when(k == nk - 1)
    def _():
        push = acc_ref[...] + bp_ref[...]
        mean = jnp.mean(push, axis=1, keepdims=True)
        cent = push - mean
        var = jnp.mean(cent * cent, axis=1, keepdims=True)
        normed = cent / jnp.sqrt(var + EPS) * g_ref[...] + be_ref[...]

        logit = gacc_ref[...] + bg_ref[...]
        is_pop = logit > 0.0
        ptr = ptr_ref[...].astype(jnp.int32)
        can_pop = is_pop & (ptr > 0)
        can_push = jnp.logical_not(is_pop) & (ptr < DEPTH)

        m = pl.program_id(0)
        row = m * BM + lax.broadcasted_iota(jnp.int32, (BM, 1), 0)

        top_ref[...] = jnp.where(can_push, normed, 0.0)
        nptr_ref[...] = jnp.where(
            can_pop, ptr - 1, jnp.where(can_push, ptr + 1, ptr)
        ).astype(jnp.float32)
        wslot_ref[...] = jnp.where(can_push, ptr, -1)
        do_gather = jnp.logical_not(can_push) & (ptr > 0)
        gidx = DEPTH * row + jnp.clip(ptr - 1, 0, DEPTH - 1)
        gsafe_ref[...] = jnp.where(do_gather, gidx, 0)
        tgt_ref[...] = jnp.where(do_gather, row, TRASH)


def _k1(hidden_state, w_push, b_push, ln_gamma, ln_beta, w_gate, b_gate,
        pointer):
    nk = HIDDEN // BK
    grid = (B // BM, nk)
    return pl.pallas_call(
        _k1_body,
        grid=grid,
        in_specs=[
            pl.BlockSpec((BM, BK), lambda m, k: (m, k)),
            pl.BlockSpec((BK, HIDDEN), lambda m, k: (k, 0)),
            pl.BlockSpec((1, HIDDEN), lambda m, k: (0, 0)),
            pl.BlockSpec((1, HIDDEN), lambda m, k: (0, 0)),
            pl.BlockSpec((1, HIDDEN), lambda m, k: (0, 0)),
            pl.BlockSpec((BK, 1), lambda m, k: (k, 0)),
            pl.BlockSpec((1, 1), lambda m, k: (0, 0)),
            pl.BlockSpec((BM, 1), lambda m, k: (m, 0)),
        ],
        out_specs=[
            pl.BlockSpec((BM, HIDDEN), lambda m, k: (m, 0)),
            pl.BlockSpec((BM, 1), lambda m, k: (m, 0)),
            pl.BlockSpec((BM, 1), lambda m, k: (m, 0)),
            pl.BlockSpec((BM, 1), lambda m, k: (m, 0)),
            pl.BlockSpec((BM, 1), lambda m, k: (m, 0)),
        ],
        out_shape=[
            jax.ShapeDtypeStruct((B, HIDDEN), jnp.float32),
            jax.ShapeDtypeStruct((B, 1), jnp.float32),
            jax.ShapeDtypeStruct((B, 1), jnp.int32),
            jax.ShapeDtypeStruct((B, 1), jnp.int32),
            jax.ShapeDtypeStruct((B, 1), jnp.int32),
        ],
        scratch_shapes=[
            pltpu.VMEM((BM, HIDDEN), jnp.float32),
            pltpu.VMEM((BM, 1), jnp.float32),
        ],
        compiler_params=pltpu.CompilerParams(
            dimension_semantics=("parallel", "arbitrary"),
            vmem_limit_bytes=67108864,
        ),
    )(hidden_state, w_push, b_push, ln_gamma, ln_beta, w_gate, b_gate,
      pointer)


def _k2_body(stack_ref, top_ref, wslot_ref, out_ref):
    ws = wslot_ref[...].reshape(BM2, 1, 1)
    slot = lax.broadcasted_iota(jnp.int32, (BM2, DEPTH, 1), 1)
    push3 = top_ref[...].reshape(BM2, 1, HIDDEN)
    out_ref[...] = jnp.where(slot == ws, push3, stack_ref[...])


def _k2(stack, top_base, wslot):
    grid = (B // BM2,)
    return pl.pallas_call(
        _k2_body,
        grid=grid,
        in_specs=[
            pl.BlockSpec((BM2, DEPTH, HIDDEN), lambda i: (i, 0, 0)),
            pl.BlockSpec((BM2, HIDDEN), lambda i: (i, 0)),
            pl.BlockSpec((BM2, 1), lambda i: (i, 0)),
        ],
        out_specs=pl.BlockSpec((BM2, DEPTH, HIDDEN), lambda i: (i, 0, 0)),
        out_shape=jax.ShapeDtypeStruct((B, DEPTH, HIDDEN), jnp.float32),
        compiler_params=pltpu.CompilerParams(
            dimension_semantics=("arbitrary",),
        ),
    )(stack, top_base, wslot)


def _k3_body(stack_hbm, base_hbm, gidx_hbm, tgt_hbm, out_hbm,
             gidx_v, tgt0, tgt1, tgt2, tgt3, buf0, buf1,
             sem_i, sem_l, sem_g, sem_s):
    wid = lax.axis_index("s") * 2 + lax.axis_index("c")
    r0 = wid * RPT
    tgts = (tgt0, tgt1, tgt2, tgt3)
    bufs = (buf0, buf1)
    nch = RPT // CHUNK

    idx_cps = [pltpu.async_copy(gidx_hbm.at[pl.ds(r0, RPT)], gidx_v, sem_i)]
    for c in range(nch):
        idx_cps.append(
            pltpu.async_copy(tgt_hbm.at[pl.ds(r0 + c * CHUNK, CHUNK)],
                             tgts[c], sem_i))
    lin = pltpu.async_copy(base_hbm.at[pl.ds(r0, RPT)],
                           out_hbm.at[pl.ds(r0, RPT)], sem_l)
    for cp in idx_cps:
        cp.wait()

    gathers = [
        pltpu.async_copy(
            stack_hbm.at[gidx_v.at[pl.ds(c * CHUNK, CHUNK)]], bufs[c], sem_g)
        for c in range(2)
    ]
    lin.wait()
    scats = [None, None]
    for c in range(nch):
        gathers[c].wait()
        scats[c % 2] = pltpu.async_copy(bufs[c % 2], out_hbm.at[tgts[c]],
                                        sem_s)
        if c + 2 < nch:
            scats[c % 2].wait()
            gathers.append(pltpu.async_copy(
                stack_hbm.at[gidx_v.at[pl.ds((c + 2) * CHUNK, CHUNK)]],
                bufs[c % 2], sem_g))
    scats[0].wait()
    scats[1].wait()


def _k3(stack_flat, top_base, gsafe, tgt):
    mesh = plsc.VectorSubcoreMesh(core_axis_name="c", subcore_axis_name="s")
    run = pl.kernel(
        _k3_body,
        out_type=jax.ShapeDtypeStruct((B_PAD, HIDDEN), jnp.float32),
        mesh=mesh,
        scratch_types=[
            pltpu.VMEM((RPT,), jnp.int32),
            pltpu.VMEM((CHUNK,), jnp.int32),
            pltpu.VMEM((CHUNK,), jnp.int32),
            pltpu.VMEM((CHUNK,), jnp.int32),
            pltpu.VMEM((CHUNK,), jnp.int32),
            pltpu.VMEM((CHUNK, HIDDEN), jnp.float32),
            pltpu.VMEM((CHUNK, HIDDEN), jnp.float32),
            pltpu.SemaphoreType.DMA,
            pltpu.SemaphoreType.DMA,
            pltpu.SemaphoreType.DMA,
            pltpu.SemaphoreType.DMA,
        ],
    )
    return run(stack_flat, top_base, gsafe, tgt)


def kernel(hidden_state, stack, pointer, W_push, b_push, W_gate, b_gate,
           ln_gamma, ln_beta):
    bp = b_push.reshape(1, HIDDEN)
    gam = ln_gamma.reshape(1, HIDDEN)
    bet = ln_beta.reshape(1, HIDDEN)
    bg = b_gate.reshape(1, 1)

    top_base, new_pointer, wslot, gsafe, tgt = _k1(
        hidden_state, W_push, bp, gam, bet, W_gate, bg, pointer)

    new_stack = _k2(stack, top_base, wslot)
    top_pad = _k3(stack.reshape(B * DEPTH, HIDDEN), top_base,
                  gsafe.reshape(B), tgt.reshape(B))
    return (new_stack, new_pointer, top_pad[:B])


# speedup vs baseline: 3.2734x; 3.2734x over previous
import jax
import jax.numpy as jnp
from jax import lax
from jax.experimental import pallas as pl
from jax.experimental.pallas import tpu as pltpu
from jax.experimental.pallas import tpu_sc as plsc

B = 1024
HIDDEN = 4096
DEPTH = 8
EPS = 1e-5

BM = 512
BK = 256
BM2 = 64
N_TILES = 32
RPT = B // N_TILES
CHUNK = 8
TRASH = B
B_PAD = B + 8


def _k1_body(h_ref, w_ref, bp_ref, g_ref, be_ref, wg_ref, bg_ref, ptr_ref,
             top_ref, nptr_ref, wslot_ref, gslot_ref,
             acc_ref, gacc_ref):
    k = pl.program_id(1)
    nk = pl.num_programs(1)

    h_bf = h_ref[...].astype(jnp.bfloat16)
    w_bf = w_ref[...].astype(jnp.bfloat16)
    part = jnp.dot(h_bf, w_bf, preferred_element_type=jnp.float32)
    wg_bf = wg_ref[...].astype(jnp.bfloat16)
    gpart = jnp.dot(h_bf, wg_bf, preferred_element_type=jnp.float32)

    @pl.when(k == 0)
    def _():
        acc_ref[...] = part
        gacc_ref[...] = gpart

    @pl.when(k != 0)
    def _():
        acc_ref[...] += part
        gacc_ref[...] += gpart

    @pl.when(k == nk - 1)
    def _():
        push = acc_ref[...] + bp_ref[...]
        mean = jnp.mean(push, axis=1, keepdims=True)
        cent = push - mean
        var = jnp.mean(cent * cent, axis=1, keepdims=True)
        normed = cent / jnp.sqrt(var + EPS) * g_ref[...] + be_ref[...]

        logit = gacc_ref[...] + bg_ref[...]
        is_pop = logit > 0.0
        ptr = ptr_ref[...].astype(jnp.int32)
        can_pop = is_pop & (ptr > 0)
        can_push = jnp.logical_not(is_pop) & (ptr < DEPTH)

        top_ref[...] = jnp.where(can_push, normed, 0.0)
        nptr_ref[...] = jnp.where(
            can_pop, ptr - 1, jnp.where(can_push, ptr + 1, ptr)
        ).astype(jnp.float32)
        wslot_ref[...] = jnp.where(can_push, ptr, -1)
        do_gather = jnp.logical_not(can_push) & (ptr > 0)
        gslot_ref[...] = jnp.where(do_gather, ptr - 1, -1)


def _k1(hidden_state, w_push, b_push, ln_gamma, ln_beta, w_gate, b_gate,
        pointer):
    nk = HIDDEN // BK
    grid = (B // BM, nk)
    return pl.pallas_call(
        _k1_body,
        grid=grid,
        in_specs=[
            pl.BlockSpec((BM, BK), lambda m, k: (m, k)),
            pl.BlockSpec((BK, HIDDEN), lambda m, k: (k, 0)),
            pl.BlockSpec((1, HIDDEN), lambda m, k: (0, 0)),
            pl.BlockSpec((1, HIDDEN), lambda m, k: (0, 0)),
            pl.BlockSpec((1, HIDDEN), lambda m, k: (0, 0)),
            pl.BlockSpec((BK, 1), lambda m, k: (k, 0)),
            pl.BlockSpec((1, 1), lambda m, k: (0, 0)),
            pl.BlockSpec((BM, 1), lambda m, k: (m, 0)),
        ],
        out_specs=[
            pl.BlockSpec((BM, HIDDEN), lambda m, k: (m, 0)),
            pl.BlockSpec((BM, 1), lambda m, k: (m, 0)),
            pl.BlockSpec((BM, 1), lambda m, k: (m, 0)),
            pl.BlockSpec((BM, 1), lambda m, k: (m, 0)),
        ],
        out_shape=[
            jax.ShapeDtypeStruct((B, HIDDEN), jnp.float32),
            jax.ShapeDtypeStruct((B, 1), jnp.float32),
            jax.ShapeDtypeStruct((B, 1), jnp.int32),
            jax.ShapeDtypeStruct((B, 1), jnp.int32),
        ],
        scratch_shapes=[
            pltpu.VMEM((BM, HIDDEN), jnp.float32),
            pltpu.VMEM((BM, 1), jnp.float32),
        ],
        compiler_params=pltpu.CompilerParams(
            dimension_semantics=("parallel", "arbitrary"),
            vmem_limit_bytes=67108864,
        ),
    )(hidden_state, w_push, b_push, ln_gamma, ln_beta, w_gate, b_gate,
      pointer)


def _k2_body(stack_ref, top_ref, wslot_ref, gslot_ref, out_ref, otop_ref):
    ws = wslot_ref[...].reshape(BM2, 1, 1)
    gs = gslot_ref[...].reshape(BM2, 1, 1)
    slot = lax.broadcasted_iota(jnp.int32, (BM2, DEPTH, 1), 1)
    stack3 = stack_ref[...]
    push3 = top_ref[...].reshape(BM2, 1, HIDDEN)
    out_ref[...] = jnp.where(slot == ws, push3, stack3)
    prev = jnp.sum(jnp.where(slot == gs, stack3, 0.0), axis=1)
    otop_ref[...] = jnp.where(gslot_ref[...] >= 0, prev, top_ref[...])


def _k2(stack, top_base, wslot, gslot):
    grid = (B // BM2,)
    return pl.pallas_call(
        _k2_body,
        grid=grid,
        in_specs=[
            pl.BlockSpec((BM2, DEPTH, HIDDEN), lambda i: (i, 0, 0)),
            pl.BlockSpec((BM2, HIDDEN), lambda i: (i, 0)),
            pl.BlockSpec((BM2, 1), lambda i: (i, 0)),
            pl.BlockSpec((BM2, 1), lambda i: (i, 0)),
        ],
        out_specs=[
            pl.BlockSpec((BM2, DEPTH, HIDDEN), lambda i: (i, 0, 0)),
            pl.BlockSpec((BM2, HIDDEN), lambda i: (i, 0)),
        ],
        out_shape=[
            jax.ShapeDtypeStruct((B, DEPTH, HIDDEN), jnp.float32),
            jax.ShapeDtypeStruct((B, HIDDEN), jnp.float32),
        ],
        compiler_params=pltpu.CompilerParams(
            dimension_semantics=("arbitrary",),
        ),
    )(stack, top_base, wslot, gslot)


def _k3_body(stack_hbm, base_hbm, gidx_hbm, tgt_hbm, out_hbm,
             gidx_v, tgt0, tgt1, tgt2, tgt3, buf0, buf1,
             sem_i, sem_l, sem_g, sem_s):
    wid = lax.axis_index("s") * 2 + lax.axis_index("c")
    r0 = wid * RPT
    tgts = (tgt0, tgt1, tgt2, tgt3)
    bufs = (buf0, buf1)
    nch = RPT // CHUNK

    idx_cps = [pltpu.async_copy(gidx_hbm.at[pl.ds(r0, RPT)], gidx_v, sem_i)]
    for c in range(nch):
        idx_cps.append(
            pltpu.async_copy(tgt_hbm.at[pl.ds(r0 + c * CHUNK, CHUNK)],
                             tgts[c], sem_i))
    lin = pltpu.async_copy(base_hbm.at[pl.ds(r0, RPT)],
                           out_hbm.at[pl.ds(r0, RPT)], sem_l)
    for cp in idx_cps:
        cp.wait()

    gathers = [
        pltpu.async_copy(
            stack_hbm.at[gidx_v.at[pl.ds(c * CHUNK, CHUNK)]], bufs[c], sem_g)
        for c in range(2)
    ]
    lin.wait()
    scats = [None, None]
    for c in range(nch):
        gathers[c].wait()
        scats[c % 2] = pltpu.async_copy(bufs[c % 2], out_hbm.at[tgts[c]],
                                        sem_s)
        if c + 2 < nch:
            scats[c % 2].wait()
            gathers.append(pltpu.async_copy(
                stack_hbm.at[gidx_v.at[pl.ds((c + 2) * CHUNK, CHUNK)]],
                bufs[c % 2], sem_g))
    scats[0].wait()
    scats[1].wait()


def _k3(stack_flat, top_base, gsafe, tgt):
    mesh = plsc.VectorSubcoreMesh(core_axis_name="c", subcore_axis_name="s")
    run = pl.kernel(
        _k3_body,
        out_type=jax.ShapeDtypeStruct((B_PAD, HIDDEN), jnp.float32),
        mesh=mesh,
        scratch_types=[
            pltpu.VMEM((RPT,), jnp.int32),
            pltpu.VMEM((CHUNK,), jnp.int32),
            pltpu.VMEM((CHUNK,), jnp.int32),
            pltpu.VMEM((CHUNK,), jnp.int32),
            pltpu.VMEM((CHUNK,), jnp.int32),
            pltpu.VMEM((CHUNK, HIDDEN), jnp.float32),
            pltpu.VMEM((CHUNK, HIDDEN), jnp.float32),
            pltpu.SemaphoreType.DMA,
            pltpu.SemaphoreType.DMA,
            pltpu.SemaphoreType.DMA,
            pltpu.SemaphoreType.DMA,
        ],
    )
    return run(stack_flat, top_base, gsafe, tgt)


def kernel(hidden_state, stack, pointer, W_push, b_push, W_gate, b_gate,
           ln_gamma, ln_beta):
    bp = b_push.reshape(1, HIDDEN)
    gam = ln_gamma.reshape(1, HIDDEN)
    bet = ln_beta.reshape(1, HIDDEN)
    bg = b_gate.reshape(1, 1)

    top_base, new_pointer, wslot, gslot = _k1(
        hidden_state, W_push, bp, gam, bet, W_gate, bg, pointer)

    new_stack, top_item = _k2(stack, top_base, wslot, gslot)
    return (new_stack, new_pointer, top_item)


# speedup vs baseline: 3.7635x; 1.1497x over previous
import jax
import jax.numpy as jnp
from jax import lax
from jax.experimental import pallas as pl
from jax.experimental.pallas import tpu as pltpu
from jax.experimental.pallas import tpu_sc as plsc

B = 1024
HIDDEN = 4096
DEPTH = 8
EPS = 1e-5

BK = 512
LM = 256
BM2 = 64
N_TILES = 32
RPT = B // N_TILES
CHUNK = 8
TRASH = B
B_PAD = B + 8


def _k1_body(h_ref, w_ref, bp_ref, g_ref, be_ref, wg_ref, bg_ref, ptr_ref,
             top_ref, nptr_ref, wslot_ref, gslot_ref,
             acc_ref, gacc_ref):
    s = pl.program_id(0)
    nk = HIDDEN // BK

    @pl.when(s < nk)
    def _():
        h_bf = h_ref[...].astype(jnp.bfloat16)
        w_bf = w_ref[...].astype(jnp.bfloat16)
        part = jnp.dot(h_bf, w_bf, preferred_element_type=jnp.float32)
        wg_bf = wg_ref[...].astype(jnp.bfloat16)
        gpart = jnp.dot(h_bf, wg_bf, preferred_element_type=jnp.float32)

        @pl.when(s == 0)
        def _():
            acc_ref[...] = part
            gacc_ref[...] = gpart

        @pl.when(s != 0)
        def _():
            acc_ref[...] += part
            gacc_ref[...] += gpart

    @pl.when(s >= nk)
    def _():
        m = s - nk
        rows = pl.ds(m * LM, LM)
        push = acc_ref[rows, :] + bp_ref[...]
        mean = jnp.mean(push, axis=1, keepdims=True)
        cent = push - mean
        var = jnp.mean(cent * cent, axis=1, keepdims=True)
        normed = cent / jnp.sqrt(var + EPS) * g_ref[...] + be_ref[...]

        logit = gacc_ref[rows, :] + bg_ref[...]
        is_pop = logit > 0.0
        ptr = ptr_ref[...].astype(jnp.int32)
        can_pop = is_pop & (ptr > 0)
        can_push = jnp.logical_not(is_pop) & (ptr < DEPTH)

        top_ref[...] = jnp.where(can_push, normed, 0.0)
        nptr_ref[...] = jnp.where(
            can_pop, ptr - 1, jnp.where(can_push, ptr + 1, ptr)
        ).astype(jnp.float32)
        wslot_ref[...] = jnp.where(can_push, ptr, -1)
        do_gather = jnp.logical_not(can_push) & (ptr > 0)
        gslot_ref[...] = jnp.where(do_gather, ptr - 1, -1)


def _k1(hidden_state, w_push, b_push, ln_gamma, ln_beta, w_gate, b_gate,
        pointer):
    nk = HIDDEN // BK
    nm = B // LM
    kc = nk - 1
    grid = (nk + nm,)

    def mblk(st):
        return (jnp.maximum(st - nk, 0), 0)

    return pl.pallas_call(
        _k1_body,
        grid=grid,
        in_specs=[
            pl.BlockSpec((B, BK), lambda st: (0, jnp.minimum(st, kc))),
            pl.BlockSpec((BK, HIDDEN), lambda st: (jnp.minimum(st, kc), 0)),
            pl.BlockSpec((1, HIDDEN), lambda st: (0, 0)),
            pl.BlockSpec((1, HIDDEN), lambda st: (0, 0)),
            pl.BlockSpec((1, HIDDEN), lambda st: (0, 0)),
            pl.BlockSpec((BK, 1), lambda st: (jnp.minimum(st, kc), 0)),
            pl.BlockSpec((1, 1), lambda st: (0, 0)),
            pl.BlockSpec((LM, 1), mblk),
        ],
        out_specs=[
            pl.BlockSpec((LM, HIDDEN), mblk),
            pl.BlockSpec((LM, 1), mblk),
            pl.BlockSpec((LM, 1), mblk),
            pl.BlockSpec((LM, 1), mblk),
        ],
        out_shape=[
            jax.ShapeDtypeStruct((B, HIDDEN), jnp.float32),
            jax.ShapeDtypeStruct((B, 1), jnp.float32),
            jax.ShapeDtypeStruct((B, 1), jnp.int32),
            jax.ShapeDtypeStruct((B, 1), jnp.int32),
        ],
        scratch_shapes=[
            pltpu.VMEM((B, HIDDEN), jnp.float32),
            pltpu.VMEM((B, 1), jnp.float32),
        ],
        compiler_params=pltpu.CompilerParams(
            dimension_semantics=("arbitrary",),
            vmem_limit_bytes=67108864,
        ),
    )(hidden_state, w_push, b_push, ln_gamma, ln_beta, w_gate, b_gate,
      pointer)


def _k2_body(stack_ref, top_ref, wslot_ref, gslot_ref, out_ref, otop_ref):
    ws = wslot_ref[...].reshape(BM2, 1, 1)
    gs = gslot_ref[...].reshape(BM2, 1, 1)
    slot = lax.broadcasted_iota(jnp.int32, (BM2, DEPTH, 1), 1)
    stack3 = stack_ref[...]
    push3 = top_ref[...].reshape(BM2, 1, HIDDEN)
    out_ref[...] = jnp.where(slot == ws, push3, stack3)
    prev = jnp.sum(jnp.where(slot == gs, stack3, 0.0), axis=1)
    otop_ref[...] = jnp.where(gslot_ref[...] >= 0, prev, top_ref[...])


def _k2(stack, top_base, wslot, gslot):
    grid = (B // BM2,)
    return pl.pallas_call(
        _k2_body,
        grid=grid,
        in_specs=[
            pl.BlockSpec((BM2, DEPTH, HIDDEN), lambda i: (i, 0, 0)),
            pl.BlockSpec((BM2, HIDDEN), lambda i: (i, 0)),
            pl.BlockSpec((BM2, 1), lambda i: (i, 0)),
            pl.BlockSpec((BM2, 1), lambda i: (i, 0)),
        ],
        out_specs=[
            pl.BlockSpec((BM2, DEPTH, HIDDEN), lambda i: (i, 0, 0)),
            pl.BlockSpec((BM2, HIDDEN), lambda i: (i, 0)),
        ],
        out_shape=[
            jax.ShapeDtypeStruct((B, DEPTH, HIDDEN), jnp.float32),
            jax.ShapeDtypeStruct((B, HIDDEN), jnp.float32),
        ],
        compiler_params=pltpu.CompilerParams(
            dimension_semantics=("arbitrary",),
        ),
    )(stack, top_base, wslot, gslot)


def _k3_body(stack_hbm, base_hbm, gidx_hbm, tgt_hbm, out_hbm,
             gidx_v, tgt0, tgt1, tgt2, tgt3, buf0, buf1,
             sem_i, sem_l, sem_g, sem_s):
    wid = lax.axis_index("s") * 2 + lax.axis_index("c")
    r0 = wid * RPT
    tgts = (tgt0, tgt1, tgt2, tgt3)
    bufs = (buf0, buf1)
    nch = RPT // CHUNK

    idx_cps = [pltpu.async_copy(gidx_hbm.at[pl.ds(r0, RPT)], gidx_v, sem_i)]
    for c in range(nch):
        idx_cps.append(
            pltpu.async_copy(tgt_hbm.at[pl.ds(r0 + c * CHUNK, CHUNK)],
                             tgts[c], sem_i))
    lin = pltpu.async_copy(base_hbm.at[pl.ds(r0, RPT)],
                           out_hbm.at[pl.ds(r0, RPT)], sem_l)
    for cp in idx_cps:
        cp.wait()

    gathers = [
        pltpu.async_copy(
            stack_hbm.at[gidx_v.at[pl.ds(c * CHUNK, CHUNK)]], bufs[c], sem_g)
        for c in range(2)
    ]
    lin.wait()
    scats = [None, None]
    for c in range(nch):
        gathers[c].wait()
        scats[c % 2] = pltpu.async_copy(bufs[c % 2], out_hbm.at[tgts[c]],
                                        sem_s)
        if c + 2 < nch:
            scats[c % 2].wait()
            gathers.append(pltpu.async_copy(
                stack_hbm.at[gidx_v.at[pl.ds((c + 2) * CHUNK, CHUNK)]],
                bufs[c % 2], sem_g))
    scats[0].wait()
    scats[1].wait()


def _k3(stack_flat, top_base, gsafe, tgt):
    mesh = plsc.VectorSubcoreMesh(core_axis_name="c", subcore_axis_name="s")
    run = pl.kernel(
        _k3_body,
        out_type=jax.ShapeDtypeStruct((B_PAD, HIDDEN), jnp.float32),
        mesh=mesh,
        scratch_types=[
            pltpu.VMEM((RPT,), jnp.int32),
            pltpu.VMEM((CHUNK,), jnp.int32),
            pltpu.VMEM((CHUNK,), jnp.int32),
            pltpu.VMEM((CHUNK,), jnp.int32),
            pltpu.VMEM((CHUNK,), jnp.int32),
            pltpu.VMEM((CHUNK, HIDDEN), jnp.float32),
            pltpu.VMEM((CHUNK, HIDDEN), jnp.float32),
            pltpu.SemaphoreType.DMA,
            pltpu.SemaphoreType.DMA,
            pltpu.SemaphoreType.DMA,
            pltpu.SemaphoreType.DMA,
        ],
    )
    return run(stack_flat, top_base, gsafe, tgt)


def kernel(hidden_state, stack, pointer, W_push, b_push, W_gate, b_gate,
           ln_gamma, ln_beta):
    bp = b_push.reshape(1, HIDDEN)
    gam = ln_gamma.reshape(1, HIDDEN)
    bet = ln_beta.reshape(1, HIDDEN)
    bg = b_gate.reshape(1, 1)

    top_base, new_pointer, wslot, gslot = _k1(
        hidden_state, W_push, bp, gam, bet, W_gate, bg, pointer)

    new_stack, top_item = _k2(stack, top_base, wslot, gslot)
    return (new_stack, new_pointer, top_item)
